# baseline (speedup 1.0000x reference)
import jax
import jax.numpy as jnp
import numpy as np
from jax import lax
from jax.experimental import pallas as pl
from jax.experimental.pallas import tpu as pltpu
from jax.experimental.pallas import tpu_sc as plsc

PATCH = 12
K_NEIGH = 3
NC = 2
NS = 16

_GATHER_DNUMS = lax.GatherDimensionNumbers(
    offset_dims=(), collapsed_slice_dims=(0,), start_index_map=(0,))


def _shuffle_reduce(v, combine):
    lane = lax.iota(jnp.int32, 16)
    for step in (1, 2, 4, 8):
        idx = jnp.bitwise_xor(lane, step).reshape(16, 1)
        peer = lax.gather(v, idx, _GATHER_DNUMS, slice_sizes=(1,),
                          mode=lax.GatherScatterMode.PROMISE_IN_BOUNDS)
        v = combine(v, peer)
    return v


def _sc_body(au_hbm, am_hbm, av_hbm, g_hbm, samp_hbm, a_v, g_v, samp_v):
    n_nodes = au_hbm.shape[0]
    wid = lax.axis_index("s") * NC + lax.axis_index("c")

    n = wid
    pltpu.sync_copy(au_hbm.at[n], a_v.at[0])
    pltpu.sync_copy(am_hbm.at[n], a_v.at[1])
    pltpu.sync_copy(av_hbm.at[n], a_v.at[2])
    for j in range(K_NEIGH):
        pltpu.sync_copy(g_hbm.at[j, n], g_v.at[j])

    halves = n_nodes // 16
    a_h = []
    for h in range(halves):
        u = a_v[0, pl.ds(16 * h, 16)]
        am = a_v[1, pl.ds(16 * h, 16)]
        av = a_v[2, pl.ds(16 * h, 16)]
        a_h.append(u * am + av)

    lane = lax.iota(jnp.int32, 16)
    sampvec = jnp.zeros((16,), jnp.int32)
    for j in range(K_NEIGH):
        w_h = [a_h[h] * jnp.exp(g_v[j, pl.ds(16 * h, 16)])
               for h in range(halves)]
        m = w_h[0]
        for h in range(1, halves):
            m = jnp.maximum(m, w_h[h])
        m = _shuffle_reduce(m, jnp.maximum)
        cand = jnp.full((16,), 2 * n_nodes, jnp.int32)
        for h in range(halves):
            cand = jnp.minimum(
                cand, jnp.where(w_h[h] == m, lane + 16 * h, 2 * n_nodes))
        idx = _shuffle_reduce(cand, jnp.minimum)
        sampvec = jnp.where(lane == j, idx, sampvec)
    samp_v[...] = sampvec
    pltpu.sync_copy(samp_v, samp_hbm.at[n])


def _tc_body(samp_ref, ht_ref, w_ref, b_ref, out_ref):
    n_nodes = ht_ref.shape[1]
    for n in range(n_nodes):
        acc = b_ref[...]
        for c in range(K_NEIGH + 1):
            s = n if c == 0 else samp_ref[n, c - 1]
            xr = ht_ref[0, s]
            acc = acc + jax.lax.dot_general(
                w_ref[c], xr,
                dimension_numbers=(((1,), (1,)), ((), ())),
                preferred_element_type=jnp.float32,
            )
        out_ref[0, n] = acc


def kernel(long_term_history, W, b, adj_mx, adj_u, adj_v):
    Bsz, N, C, T = long_term_history.shape
    P = T // PATCH
    E = W.shape[0]

    gumb = jax.random.gumbel(jax.random.key(42), (K_NEIGH, N, N), jnp.float32)

    mesh = plsc.VectorSubcoreMesh(core_axis_name="c", subcore_axis_name="s")
    sc_fn = pl.kernel(
        _sc_body,
        out_type=[jax.ShapeDtypeStruct((N, 16), jnp.int32)],
        mesh=mesh,
        scratch_types=[
            pltpu.VMEM((3, N), jnp.float32),
            pltpu.VMEM((K_NEIGH, N), jnp.float32),
            pltpu.VMEM((16,), jnp.int32),
        ],
    )
    (sampled,) = sc_fn(adj_u, adj_mx, adj_v, gumb)

    hist = long_term_history.reshape(Bsz, N, P, PATCH)
    w3 = jnp.transpose(W, (1, 0, 2))
    b2 = b.reshape(E, 1)

    out = pl.pallas_call(
        _tc_body,
        grid=(Bsz,),
        in_specs=[
            pl.BlockSpec(memory_space=pltpu.SMEM),
            pl.BlockSpec((1, N, P, PATCH), lambda i: (i, 0, 0, 0)),
            pl.BlockSpec((K_NEIGH + 1, E, PATCH), lambda i: (0, 0, 0)),
            pl.BlockSpec((E, 1), lambda i: (0, 0)),
        ],
        out_specs=pl.BlockSpec((1, N, E, P), lambda i: (i, 0, 0, 0)),
        out_shape=jax.ShapeDtypeStruct((Bsz, N, E, P), jnp.float32),
        compiler_params=pltpu.CompilerParams(
            dimension_semantics=("parallel",),
        ),
    )(sampled, hist, w3, b2)
    return out


# speedup vs baseline: 1.3092x; 1.3092x over previous
import jax
import jax.numpy as jnp
import numpy as np
from jax import lax
from jax.experimental import pallas as pl
from jax.experimental.pallas import tpu as pltpu
from jax.experimental.pallas import tpu_sc as plsc

PATCH = 12
K_NEIGH = 3
NC = 2
NS = 16

_GATHER_DNUMS = lax.GatherDimensionNumbers(
    offset_dims=(), collapsed_slice_dims=(0,), start_index_map=(0,))


def _shuffle_reduce(v, combine):
    lane = lax.iota(jnp.int32, 16)
    for step in (1, 2, 4, 8):
        idx = jnp.bitwise_xor(lane, step).reshape(16, 1)
        peer = lax.gather(v, idx, _GATHER_DNUMS, slice_sizes=(1,),
                          mode=lax.GatherScatterMode.PROMISE_IN_BOUNDS)
        v = combine(v, peer)
    return v


def _sc_body(au_hbm, am_hbm, av_hbm, g_hbm, samp_hbm, a_v, g_v, samp_v):
    n_nodes = au_hbm.shape[0]
    wid = lax.axis_index("s") * NC + lax.axis_index("c")

    n = wid
    pltpu.sync_copy(au_hbm.at[n], a_v.at[0])
    pltpu.sync_copy(am_hbm.at[n], a_v.at[1])
    pltpu.sync_copy(av_hbm.at[n], a_v.at[2])
    for j in range(K_NEIGH):
        pltpu.sync_copy(g_hbm.at[j, n], g_v.at[j])

    halves = n_nodes // 16
    a_h = []
    for h in range(halves):
        u = a_v[0, pl.ds(16 * h, 16)]
        am = a_v[1, pl.ds(16 * h, 16)]
        av = a_v[2, pl.ds(16 * h, 16)]
        a_h.append(u * am + av)

    lane = lax.iota(jnp.int32, 16)
    sampvec = jnp.zeros((16,), jnp.int32)
    for j in range(K_NEIGH):
        w_h = [a_h[h] * jnp.exp(g_v[j, pl.ds(16 * h, 16)])
               for h in range(halves)]
        m = w_h[0]
        for h in range(1, halves):
            m = jnp.maximum(m, w_h[h])
        m = _shuffle_reduce(m, jnp.maximum)
        cand = jnp.full((16,), 2 * n_nodes, jnp.int32)
        for h in range(halves):
            cand = jnp.minimum(
                cand, jnp.where(w_h[h] == m, lane + 16 * h, 2 * n_nodes))
        idx = _shuffle_reduce(cand, jnp.minimum)
        sampvec = jnp.where(lane == j, idx, sampvec)
    samp_v[...] = sampvec
    pltpu.sync_copy(samp_v, samp_hbm.at[n])


LPAD = 16


def _tc_body(samp_ref, ht_ref, w_ref, b_ref, out_ref):
    n_nodes = ht_ref.shape[1]
    for n in range(n_nodes):
        rows = [ht_ref[0, n]]
        for j in range(K_NEIGH):
            s = samp_ref[n, j]
            rows.append(ht_ref[0, s])
        x = jnp.concatenate(rows, axis=0)
        acc = jax.lax.dot_general(
            w_ref[...], x,
            dimension_numbers=(((1,), (0,)), ((), ())),
            preferred_element_type=jnp.float32,
        )
        out_ref[0, n] = acc + b_ref[...]


def kernel(long_term_history, W, b, adj_mx, adj_u, adj_v):
    Bsz, N, C, T = long_term_history.shape
    P = T // PATCH
    E = W.shape[0]

    gumb = jax.random.gumbel(jax.random.key(42), (K_NEIGH, N, N), jnp.float32)

    mesh = plsc.VectorSubcoreMesh(core_axis_name="c", subcore_axis_name="s")
    sc_fn = pl.kernel(
        _sc_body,
        out_type=[jax.ShapeDtypeStruct((N, 16), jnp.int32)],
        mesh=mesh,
        scratch_types=[
            pltpu.VMEM((3, N), jnp.float32),
            pltpu.VMEM((K_NEIGH, N), jnp.float32),
            pltpu.VMEM((16,), jnp.int32),
        ],
    )
    (sampled,) = sc_fn(adj_u, adj_mx, adj_v, gumb)

    hist4 = long_term_history.reshape(Bsz, N, P, PATCH)
    eye = jnp.eye(PATCH, LPAD, dtype=jnp.float32)
    ht = jnp.einsum('bnpl,lk->bnkp', hist4, eye,
                    preferred_element_type=jnp.float32)

    wp = jnp.pad(W, ((0, 0), (0, 0), (0, LPAD - PATCH)))
    wp = wp.reshape(E, (K_NEIGH + 1) * LPAD)
    b2 = b.reshape(E, 1)

    out = pl.pallas_call(
        _tc_body,
        grid=(Bsz,),
        in_specs=[
            pl.BlockSpec(memory_space=pltpu.SMEM),
            pl.BlockSpec((1, N, LPAD, P), lambda i: (i, 0, 0, 0)),
            pl.BlockSpec((E, (K_NEIGH + 1) * LPAD), lambda i: (0, 0)),
            pl.BlockSpec((E, 1), lambda i: (0, 0)),
        ],
        out_specs=pl.BlockSpec((1, N, E, P), lambda i: (i, 0, 0, 0)),
        out_shape=jax.ShapeDtypeStruct((Bsz, N, E, P), jnp.float32),
        compiler_params=pltpu.CompilerParams(
            dimension_semantics=("parallel",),
        ),
    )(sampled, ht, wp, b2)
    return out


# speedup vs baseline: 1.5320x; 1.1702x over previous
import jax
import jax.numpy as jnp
import numpy as np
from jax import lax
from jax.experimental import pallas as pl
from jax.experimental.pallas import tpu as pltpu
from jax.experimental.pallas import tpu_sc as plsc

PATCH = 12
K_NEIGH = 3
NC = 2
NS = 16

_GATHER_DNUMS = lax.GatherDimensionNumbers(
    offset_dims=(), collapsed_slice_dims=(0,), start_index_map=(0,))


def _shuffle_reduce(v, combine):
    lane = lax.iota(jnp.int32, 16)
    for step in (1, 2, 4, 8):
        idx = jnp.bitwise_xor(lane, step).reshape(16, 1)
        peer = lax.gather(v, idx, _GATHER_DNUMS, slice_sizes=(1,),
                          mode=lax.GatherScatterMode.PROMISE_IN_BOUNDS)
        v = combine(v, peer)
    return v


def _sc_body(au_hbm, am_hbm, av_hbm, g_hbm, samp_hbm, a_v, g_v, samp_v):
    n_nodes = au_hbm.shape[0]
    wid = lax.axis_index("s") * NC + lax.axis_index("c")

    n = wid
    pltpu.sync_copy(au_hbm.at[n], a_v.at[0])
    pltpu.sync_copy(am_hbm.at[n], a_v.at[1])
    pltpu.sync_copy(av_hbm.at[n], a_v.at[2])
    for j in range(K_NEIGH):
        pltpu.sync_copy(g_hbm.at[j, n], g_v.at[j])

    halves = n_nodes // 16
    a_h = []
    for h in range(halves):
        u = a_v[0, pl.ds(16 * h, 16)]
        am = a_v[1, pl.ds(16 * h, 16)]
        av = a_v[2, pl.ds(16 * h, 16)]
        a_h.append(u * am + av)

    lane = lax.iota(jnp.int32, 16)
    sampvec = jnp.zeros((16,), jnp.int32)
    for j in range(K_NEIGH):
        w_h = [a_h[h] * jnp.exp(g_v[j, pl.ds(16 * h, 16)])
               for h in range(halves)]
        m = w_h[0]
        for h in range(1, halves):
            m = jnp.maximum(m, w_h[h])
        m = _shuffle_reduce(m, jnp.maximum)
        cand = jnp.full((16,), 2 * n_nodes, jnp.int32)
        for h in range(halves):
            cand = jnp.minimum(
                cand, jnp.where(w_h[h] == m, lane + 16 * h, 2 * n_nodes))
        idx = _shuffle_reduce(cand, jnp.minimum)
        sampvec = jnp.where(lane == j, idx, sampvec)
    samp_v[...] = sampvec
    pltpu.sync_copy(samp_v, samp_hbm.at[n])


LPAD = 16


def _tc_body(samp_ref, ht_ref, w_ref, b_ref, out_ref):
    n_nodes = ht_ref.shape[1]
    for n in range(n_nodes):
        rows = [ht_ref[0, n]]
        for j in range(K_NEIGH):
            s = samp_ref[n, j]
            rows.append(ht_ref[0, s])
        x = jnp.concatenate(rows, axis=0)
        acc = jax.lax.dot_general(
            w_ref[...], x,
            dimension_numbers=(((1,), (0,)), ((), ())),
            preferred_element_type=jnp.float32,
        )
        out_ref[0, n] = acc + b_ref[...]


def kernel(long_term_history, W, b, adj_mx, adj_u, adj_v):
    Bsz, N, C, T = long_term_history.shape
    P = T // PATCH
    E = W.shape[0]

    gumb = jax.random.gumbel(jax.random.key(42), (K_NEIGH, N, N), jnp.float32)

    mesh = plsc.VectorSubcoreMesh(core_axis_name="c", subcore_axis_name="s")
    sc_fn = pl.kernel(
        _sc_body,
        out_type=[jax.ShapeDtypeStruct((N, 16), jnp.int32)],
        mesh=mesh,
        scratch_types=[
            pltpu.VMEM((3, N), jnp.float32),
            pltpu.VMEM((K_NEIGH, N), jnp.float32),
            pltpu.VMEM((16,), jnp.int32),
        ],
    )
    (sampled,) = sc_fn(adj_u, adj_mx, adj_v, gumb)

    hist4 = long_term_history.reshape(Bsz, N, P, PATCH)
    ht = jnp.transpose(hist4, (0, 1, 3, 2))
    ht = jnp.pad(ht, ((0, 0), (0, 0), (0, LPAD - PATCH), (0, 0)))

    wp = jnp.pad(W, ((0, 0), (0, 0), (0, LPAD - PATCH)))
    wp = wp.reshape(E, (K_NEIGH + 1) * LPAD)
    b2 = b.reshape(E, 1)

    out = pl.pallas_call(
        _tc_body,
        grid=(Bsz,),
        in_specs=[
            pl.BlockSpec(memory_space=pltpu.SMEM),
            pl.BlockSpec((1, N, LPAD, P), lambda i: (i, 0, 0, 0)),
            pl.BlockSpec((E, (K_NEIGH + 1) * LPAD), lambda i: (0, 0)),
            pl.BlockSpec((E, 1), lambda i: (0, 0)),
        ],
        out_specs=pl.BlockSpec((1, N, E, P), lambda i: (i, 0, 0, 0)),
        out_shape=jax.ShapeDtypeStruct((Bsz, N, E, P), jnp.float32),
        compiler_params=pltpu.CompilerParams(
            dimension_semantics=("parallel",),
        ),
    )(sampled, ht, wp, b2)
    return out


# speedup vs baseline: 1.5350x; 1.0020x over previous
import jax
import jax.numpy as jnp
import numpy as np
from jax import lax
from jax.experimental import pallas as pl
from jax.experimental.pallas import tpu as pltpu
from jax.experimental.pallas import tpu_sc as plsc

PATCH = 12
K_NEIGH = 3
NC = 2
NS = 16

_GATHER_DNUMS = lax.GatherDimensionNumbers(
    offset_dims=(), collapsed_slice_dims=(0,), start_index_map=(0,))


def _shuffle_reduce(v, combine):
    lane = lax.iota(jnp.int32, 16)
    for step in (1, 2, 4, 8):
        idx = jnp.bitwise_xor(lane, step).reshape(16, 1)
        peer = lax.gather(v, idx, _GATHER_DNUMS, slice_sizes=(1,),
                          mode=lax.GatherScatterMode.PROMISE_IN_BOUNDS)
        v = combine(v, peer)
    return v


def _sc_body(au_hbm, am_hbm, av_hbm, g_hbm, samp_hbm, a_v, g_v, samp_v):
    n_nodes = a_v.shape[1]
    wid = lax.axis_index("s") * NC + lax.axis_index("c")

    n = wid
    pltpu.sync_copy(au_hbm.at[pl.ds(n * n_nodes, n_nodes)], a_v.at[0])
    pltpu.sync_copy(am_hbm.at[pl.ds(n * n_nodes, n_nodes)], a_v.at[1])
    pltpu.sync_copy(av_hbm.at[pl.ds(n * n_nodes, n_nodes)], a_v.at[2])
    for j in range(K_NEIGH):
        pltpu.sync_copy(
            g_hbm.at[pl.ds((j * n_nodes + n) * n_nodes, n_nodes)], g_v.at[j])

    halves = n_nodes // 16
    a_h = []
    for h in range(halves):
        u = a_v[0, pl.ds(16 * h, 16)]
        am = a_v[1, pl.ds(16 * h, 16)]
        av = a_v[2, pl.ds(16 * h, 16)]
        a_h.append(u * am + av)

    lane = lax.iota(jnp.int32, 16)
    sampvec = jnp.zeros((16,), jnp.int32)
    for j in range(K_NEIGH):
        w_h = [a_h[h] * jnp.exp(g_v[j, pl.ds(16 * h, 16)])
               for h in range(halves)]
        m = w_h[0]
        for h in range(1, halves):
            m = jnp.maximum(m, w_h[h])
        m = _shuffle_reduce(m, jnp.maximum)
        cand = jnp.full((16,), 2 * n_nodes, jnp.int32)
        for h in range(halves):
            cand = jnp.minimum(
                cand, jnp.where(w_h[h] == m, lane + 16 * h, 2 * n_nodes))
        idx = _shuffle_reduce(cand, jnp.minimum)
        sampvec = jnp.where(lane == j, idx, sampvec)
    samp_v[...] = sampvec
    pltpu.sync_copy(samp_v, samp_hbm.at[pl.ds(n * 16, 16)])


LPAD = 16


def _tc_body(samp_ref, ht_ref, w_ref, b_ref, out_ref):
    n_nodes = ht_ref.shape[1]
    for n in range(n_nodes):
        rows = [ht_ref[0, n]]
        for j in range(K_NEIGH):
            s = samp_ref[n * 16 + j]
            rows.append(ht_ref[0, s])
        x = jnp.concatenate(rows, axis=0)
        acc = jax.lax.dot_general(
            w_ref[...], x,
            dimension_numbers=(((1,), (0,)), ((), ())),
            preferred_element_type=jnp.float32,
        )
        out_ref[0, n] = acc + b_ref[...]


def kernel(long_term_history, W, b, adj_mx, adj_u, adj_v):
    Bsz, N, C, T = long_term_history.shape
    P = T // PATCH
    E = W.shape[0]

    gumb = jax.random.gumbel(
        jax.random.key(42), (K_NEIGH, N, N), jnp.float32).reshape(-1)

    mesh = plsc.VectorSubcoreMesh(core_axis_name="c", subcore_axis_name="s")
    sc_fn = pl.kernel(
        _sc_body,
        out_type=[jax.ShapeDtypeStruct((N * 16,), jnp.int32)],
        mesh=mesh,
        scratch_types=[
            pltpu.VMEM((3, N), jnp.float32),
            pltpu.VMEM((K_NEIGH, N), jnp.float32),
            pltpu.VMEM((16,), jnp.int32),
        ],
    )
    (sampled,) = sc_fn(adj_u.reshape(-1), adj_mx.reshape(-1),
                       adj_v.reshape(-1), gumb)

    hist4 = long_term_history.reshape(Bsz, N, P, PATCH)
    ht = jnp.transpose(hist4, (0, 1, 3, 2))
    ht = jnp.pad(ht, ((0, 0), (0, 0), (0, LPAD - PATCH), (0, 0)))

    wp = jnp.pad(W, ((0, 0), (0, 0), (0, LPAD - PATCH)))
    wp = wp.reshape(E, (K_NEIGH + 1) * LPAD)
    b2 = b.reshape(E, 1)

    out = pl.pallas_call(
        _tc_body,
        grid=(Bsz,),
        in_specs=[
            pl.BlockSpec(memory_space=pltpu.SMEM),
            pl.BlockSpec((1, N, LPAD, P), lambda i: (i, 0, 0, 0)),
            pl.BlockSpec((E, (K_NEIGH + 1) * LPAD), lambda i: (0, 0)),
            pl.BlockSpec((E, 1), lambda i: (0, 0)),
        ],
        out_specs=pl.BlockSpec((1, N, E, P), lambda i: (i, 0, 0, 0)),
        out_shape=jax.ShapeDtypeStruct((Bsz, N, E, P), jnp.float32),
        compiler_params=pltpu.CompilerParams(
            dimension_semantics=("parallel",),
        ),
    )(sampled, ht, wp, b2)
    return out


# speedup vs baseline: 2.7198x; 1.7718x over previous
import jax
import jax.numpy as jnp
import numpy as np
from jax import lax
from jax.experimental import pallas as pl
from jax.experimental.pallas import tpu as pltpu
from jax.experimental.pallas import tpu_sc as plsc

PATCH = 12
K_NEIGH = 3
NC = 2
NS = 16

_GATHER_DNUMS = lax.GatherDimensionNumbers(
    offset_dims=(), collapsed_slice_dims=(0,), start_index_map=(0,))


def _shuffle_reduce(v, combine):
    lane = lax.iota(jnp.int32, 16)
    for step in (1, 2, 4, 8):
        idx = jnp.bitwise_xor(lane, step).reshape(16, 1)
        peer = lax.gather(v, idx, _GATHER_DNUMS, slice_sizes=(1,),
                          mode=lax.GatherScatterMode.PROMISE_IN_BOUNDS)
        v = combine(v, peer)
    return v


def _sc_body(au_hbm, am_hbm, av_hbm, g_hbm, samp_hbm, a_v, g_v, samp_v):
    n_nodes = a_v.shape[1]
    wid = lax.axis_index("s") * NC + lax.axis_index("c")

    n = wid
    pltpu.sync_copy(au_hbm.at[pl.ds(n * n_nodes, n_nodes)], a_v.at[0])
    pltpu.sync_copy(am_hbm.at[pl.ds(n * n_nodes, n_nodes)], a_v.at[1])
    pltpu.sync_copy(av_hbm.at[pl.ds(n * n_nodes, n_nodes)], a_v.at[2])
    for j in range(K_NEIGH):
        pltpu.sync_copy(
            g_hbm.at[pl.ds((j * n_nodes + n) * n_nodes, n_nodes)], g_v.at[j])

    halves = n_nodes // 16
    a_h = []
    for h in range(halves):
        u = a_v[0, pl.ds(16 * h, 16)]
        am = a_v[1, pl.ds(16 * h, 16)]
        av = a_v[2, pl.ds(16 * h, 16)]
        a_h.append(u * am + av)

    lane = lax.iota(jnp.int32, 16)
    sampvec = jnp.zeros((16,), jnp.int32)
    for j in range(K_NEIGH):
        w_h = [a_h[h] * jnp.exp(g_v[j, pl.ds(16 * h, 16)])
               for h in range(halves)]
        m = w_h[0]
        for h in range(1, halves):
            m = jnp.maximum(m, w_h[h])
        m = _shuffle_reduce(m, jnp.maximum)
        cand = jnp.full((16,), 2 * n_nodes, jnp.int32)
        for h in range(halves):
            cand = jnp.minimum(
                cand, jnp.where(w_h[h] == m, lane + 16 * h, 2 * n_nodes))
        idx = _shuffle_reduce(cand, jnp.minimum)
        sampvec = jnp.where(lane == j, idx, sampvec)
    samp_v[...] = sampvec
    pltpu.sync_copy(samp_v, samp_hbm.at[pl.ds(n * 16, 16)])


LPAD = 16


def _tc_body(samp_ref, ht_ref, w_ref, b_ref, out_ref):
    n_nodes = ht_ref.shape[1]
    for n in range(n_nodes):
        rows = [ht_ref[0, n]]
        for j in range(K_NEIGH):
            s = samp_ref[n * 16 + j]
            rows.append(ht_ref[0, s])
        x = jnp.concatenate(rows, axis=0)
        acc = jax.lax.dot_general(
            x, w_ref[...],
            dimension_numbers=(((0,), (1,)), ((), ())),
            preferred_element_type=jnp.float32,
        )
        out_ref[0, n] = acc + b_ref[...]


def kernel(long_term_history, W, b, adj_mx, adj_u, adj_v):
    Bsz, N, C, T = long_term_history.shape
    P = T // PATCH
    E = W.shape[0]

    gumb = jax.random.gumbel(
        jax.random.key(42), (K_NEIGH, N, N), jnp.float32).reshape(-1)

    mesh = plsc.VectorSubcoreMesh(core_axis_name="c", subcore_axis_name="s")
    sc_fn = pl.kernel(
        _sc_body,
        out_type=[jax.ShapeDtypeStruct((N * 16,), jnp.int32)],
        mesh=mesh,
        scratch_types=[
            pltpu.VMEM((3, N), jnp.float32),
            pltpu.VMEM((K_NEIGH, N), jnp.float32),
            pltpu.VMEM((16,), jnp.int32),
        ],
    )
    (sampled,) = sc_fn(adj_u.reshape(-1), adj_mx.reshape(-1),
                       adj_v.reshape(-1), gumb)

    hist4 = long_term_history.reshape(Bsz, N, P, PATCH)
    ht = jnp.transpose(hist4, (0, 1, 3, 2))
    ht = jnp.pad(ht, ((0, 0), (0, 0), (0, LPAD - PATCH), (0, 0)))

    wp = jnp.pad(W, ((0, 0), (0, 0), (0, LPAD - PATCH)))
    wp = wp.reshape(E, (K_NEIGH + 1) * LPAD)
    b2 = b.reshape(1, E)

    out_t = pl.pallas_call(
        _tc_body,
        grid=(Bsz,),
        in_specs=[
            pl.BlockSpec(memory_space=pltpu.SMEM),
            pl.BlockSpec((1, N, LPAD, P), lambda i: (i, 0, 0, 0)),
            pl.BlockSpec((E, (K_NEIGH + 1) * LPAD), lambda i: (0, 0)),
            pl.BlockSpec((1, E), lambda i: (0, 0)),
        ],
        out_specs=pl.BlockSpec((1, N, P, E), lambda i: (i, 0, 0, 0)),
        out_shape=jax.ShapeDtypeStruct((Bsz, N, P, E), jnp.float32),
        compiler_params=pltpu.CompilerParams(
            dimension_semantics=("parallel",),
        ),
    )(sampled, ht, wp, b2)
    return jnp.transpose(out_t, (0, 1, 3, 2))
